# baseline (speedup 1.0000x reference)
import functools
import math

import numpy as np
import jax
import jax.numpy as jnp
from jax import lax
from jax.experimental import pallas as pl
from jax.experimental.pallas import tpu as pltpu
from jax.experimental.pallas import tpu_sc as plsc


def _thresh_sq() -> float:
    t = np.float32(math.pi * 3.0 / 4.0)
    x = np.float32(t * t)
    while np.float32(np.sqrt(x)) > t:
        x = np.nextafter(x, np.float32(0.0))
    while np.float32(np.sqrt(np.nextafter(x, np.float32(np.inf)))) <= t:
        x = np.nextafter(x, np.float32(np.inf))
    return float(x)


_T2 = _thresh_sq()

_NW = 32
_L = 16
_GRP = 64
_CH = 2048


def _sc_body(xi_hbm, out_hbm, buf0, buf1, out_v, sem0, sem1):
    rows_w = out_v.shape[0] * 4
    nstep = rows_w // _CH
    wid = lax.axis_index("s") * 2 + lax.axis_index("c")
    row0 = wid * rows_w
    iota64 = lax.iota(jnp.int32, _L) * 64

    bufs = (buf0, buf1)
    sems = (sem0, sem1)

    def start(step):
        src = xi_hbm.at[pl.ds((row0 + step * _CH) * 16, _CH * 16)]
        return pltpu.async_copy(src, bufs[step % 2], sems[step % 2])

    start(0)
    for step in range(nstep):
        if step + 1 < nstep:
            start(step + 1)
        buf = bufs[step % 2]
        pltpu.make_async_copy(
            xi_hbm.at[pl.ds((row0 + step * _CH) * 16, _CH * 16)],
            buf, sems[step % 2]).wait()

        def g_body(g, carry, buf=buf, step=step):
            base = g * (_GRP * 16)
            acc = jnp.zeros((_L,), jnp.int32)
            for k in range(4):
                idx = iota64 + (base + 16 * k)
                w0 = plsc.load_gather(buf, [idx])
                w1 = plsc.load_gather(buf, [idx + 1])
                w2 = plsc.load_gather(buf, [idx + 2])
                s = w0 * w0 + w1 * w1 + w2 * w2
                acc = acc | ((s > _T2).astype(jnp.int32) << (8 * k))
            out_v[pl.ds(step * (_CH // 4) + g * _L, _L)] = acc
            return carry

        lax.fori_loop(0, _CH // _GRP, g_body, 0)

    out_off = pl.multiple_of(wid * (rows_w // 4), 8)
    pltpu.sync_copy(out_v, out_hbm.at[pl.ds(out_off, rows_w // 4)])


def kernel(t, xi):
    del t
    B = xi.shape[0]
    rows_w = B // _NW
    sc_call = functools.partial(
        pl.kernel,
        mesh=plsc.VectorSubcoreMesh(core_axis_name="c", subcore_axis_name="s"),
        compiler_params=pltpu.CompilerParams(needs_layout_passes=False),
        out_type=jax.ShapeDtypeStruct((B // 4,), jnp.int32),
        scratch_types=[
            pltpu.VMEM((_CH * 16,), jnp.float32),
            pltpu.VMEM((_CH * 16,), jnp.float32),
            pltpu.VMEM((rows_w // 4,), jnp.int32),
            pltpu.SemaphoreType.DMA,
            pltpu.SemaphoreType.DMA,
        ],
    )(_sc_body)
    out_packed = sc_call(xi.reshape(-1))
    return lax.bitcast_convert_type(out_packed, jnp.uint8).reshape(B).astype(
        jnp.bool_)
